# baseline (speedup 1.0000x reference)
import functools

import jax
import jax.numpy as jnp
from jax import lax
from jax.experimental import pallas as pl
from jax.experimental.pallas import tpu as pltpu
from jax.experimental.pallas import tpu_sc as plsc

N = 10000
E = 160000
DP = 320
H = DP // 2
NLAYERS = 5
NC = 2
NS = 16
EPT = E // NS
K = 80
NCHUNK = EPT // K
NRCH = N // K
BN = 400
BE = 800

_f32 = jnp.float32


def _pad2(a, shape):
    out = jnp.zeros(shape, a.dtype)
    return lax.dynamic_update_slice(out, a, (0,) * a.ndim)


def _const_spec(shape):
    nd = len(shape)
    return pl.BlockSpec(shape, lambda *args: (0,) * nd)


def _sc_body(xh0, xh1, eah0, eah1, src, dst, out0, out1, sh, xg, eag,
             isrc, idst, sem1, sem2):
    c = lax.axis_index("c")
    s = lax.axis_index("s")

    nch = jnp.where(s < NRCH % NS, NRCH // NS + 1, NRCH // NS)

    def _zrow(r, carry):
        for i in range(H // 16):
            xg[r, pl.ds(i * 16, 16)] = jnp.zeros((16,), _f32)
        return carry
    lax.fori_loop(0, K, _zrow, 0)

    def _zchunk(q, carry):
        base = (s + NS * q) * K
        pltpu.sync_copy(xg, sh.at[pl.ds(base, K)])
        return carry
    lax.fori_loop(0, nch, _zchunk, 0)
    plsc.subcore_barrier()

    e0 = s * EPT

    def _chunk(j, carry):
        base = e0 + j * K
        pltpu.sync_copy(src.at[pl.ds(base, K)], isrc)
        pltpu.sync_copy(dst.at[pl.ds(base, K)], idst)
        esl = pl.ds(base, K)

        def _fetch(xh, eah):
            def _go():
                cp1 = pltpu.async_copy(xh.at[isrc], xg, sem1)
                cp2 = pltpu.async_copy(eah.at[esl], eag, sem2)
                cp1.wait()
                cp2.wait()
            return _go
        pl.when(c == 0)(_fetch(xh0, eah0))
        pl.when(c == 1)(_fetch(xh1, eah1))

        def _mrow(r2, inner):
            r = r2 * 2
            for i in range(H // 16):
                sl = pl.ds(i * 16, 16)
                xg[r, sl] = jnp.maximum(xg[r, sl] + eag[r, sl], 0.0)
            for i in range(H // 16):
                sl = pl.ds(i * 16, 16)
                xg[r + 1, sl] = jnp.maximum(xg[r + 1, sl] + eag[r + 1, sl],
                                            0.0)
            return inner
        lax.fori_loop(0, K // 2, _mrow, 0)
        pltpu.sync_copy(xg, sh.at[idst], add=True)
        return carry
    lax.fori_loop(0, NCHUNK, _chunk, 0)
    plsc.subcore_barrier()

    def _wchunk(q, carry):
        base = (s + NS * q) * K
        sl = pl.ds(base, K)
        pltpu.sync_copy(sh.at[sl], xg)
        pl.when(c == 0)(lambda: pltpu.sync_copy(xg, out0.at[sl]))
        pl.when(c == 1)(lambda: pltpu.sync_copy(xg, out1.at[sl]))
        return carry
    lax.fori_loop(0, nch, _wchunk, 0)


@functools.lru_cache(maxsize=1)
def _build_sc():
    mesh = plsc.VectorSubcoreMesh(
        core_axis_name="c", subcore_axis_name="s",
        num_cores=NC, num_subcores=NS)
    return pl.kernel(
        _sc_body,
        out_type=(jax.ShapeDtypeStruct((N, H), _f32),
                  jax.ShapeDtypeStruct((N, H), _f32)),
        mesh=mesh,
        scratch_types=[
            pltpu.VMEM_SHARED((N, H), _f32),
            pltpu.VMEM((K, H), _f32),
            pltpu.VMEM((K, H), _f32),
            pltpu.VMEM((K,), jnp.int32),
            pltpu.VMEM((K,), jnp.int32),
            pltpu.SemaphoreType.DMA,
            pltpu.SemaphoreType.DMA,
        ],
        compiler_params=pltpu.CompilerParams(use_tc_tiling_on_sc=False),
    )


def _sc_aggregate(xh0, xh1, eah0, eah1, src, dst):
    return _build_sc()(xh0, xh1, eah0, eah1, src, dst)


def _node_body(z_ref, ch_ref, cg_ref, at_ref, w1a_ref, w1b_ref, b1_ref,
               w2_ref, b2_ref, x0_ref, x1_ref):
    zb = z_ref[...]
    ids = lax.broadcasted_iota(jnp.int32, (BN, 128), 1)
    oh = (zb == ids).astype(_f32)
    emb = jnp.dot(oh, at_ref[...], preferred_element_type=_f32)
    t = ch_ref[...] * w1a_ref[...] + cg_ref[...] * w1b_ref[...] + b1_ref[...]
    t = jnp.maximum(t, 0.0)
    x = emb + jnp.dot(t, w2_ref[...],
                      preferred_element_type=_f32) + b2_ref[...]
    x0_ref[...] = x[:, 0:H]
    x1_ref[...] = x[:, H:DP]


def _edge_body(a0_ref, a1_ref, a2_ref, w1a_ref, w1b_ref, w1c_ref, b1_ref,
               w2_ref, b2_ref, o0_ref, o1_ref):
    t = (a0_ref[...] * w1a_ref[...] + a1_ref[...] * w1b_ref[...] +
         a2_ref[...] * w1c_ref[...] + b1_ref[...])
    t = jnp.maximum(t, 0.0)
    o = jnp.dot(t, w2_ref[...], preferred_element_type=_f32) + b2_ref[...]
    o0_ref[...] = o[:, 0:H]
    o1_ref[...] = o[:, H:DP]


def _mlp_body(x0_ref, x1_ref, a0_ref, a1_ref, w1_ref, b1_ref, w2_ref, b2_ref,
              h_ref, s1_ref, s2_ref):
    i = pl.program_id(0)
    t = jnp.dot(x0_ref[...] + a0_ref[...], w1_ref[0:H, :],
                preferred_element_type=_f32)
    t += jnp.dot(x1_ref[...] + a1_ref[...], w1_ref[H:DP, :],
                 preferred_element_type=_f32)
    t = jnp.maximum(t + b1_ref[...], 0.0)
    h = jnp.dot(t, w2_ref[...], preferred_element_type=_f32) + b2_ref[...]
    h_ref[...] = h

    @pl.when(i == 0)
    def _():
        s1_ref[...] = jnp.zeros_like(s1_ref)
        s2_ref[...] = jnp.zeros_like(s2_ref)
    s1_ref[...] += jnp.sum(h, axis=0, keepdims=True)
    s2_ref[...] += jnp.sum(h * h, axis=0, keepdims=True)


def _bn_body(h_ref, s1_ref, s2_ref, g_ref, b_ref, x0_ref, x1_ref, cs_ref):
    i = pl.program_id(0)
    mean = s1_ref[...] * (1.0 / N)
    var = s2_ref[...] * (1.0 / N) - mean * mean
    scale = g_ref[...] * lax.rsqrt(var + 1e-5)
    shift = b_ref[...] - mean * scale
    xb = jnp.maximum(h_ref[...] * scale + shift, 0.0)
    x0_ref[...] = xb[:, 0:H]
    x1_ref[...] = xb[:, H:DP]

    @pl.when(i == 0)
    def _():
        cs_ref[...] = jnp.zeros_like(cs_ref)
    cs_ref[...] += jnp.sum(xb, axis=0, keepdims=True)


def _pool_body(cs_ref, w_ref, b_ref, o_ref):
    o_ref[...] = jnp.dot(cs_ref[...] * (1.0 / N), w_ref[...],
                         preferred_element_type=_f32) + b_ref[...]


def _node_encode(z2, ch2, cg2, atp, w1a, w1b, b1, w2, b2):
    return pl.pallas_call(
        _node_body,
        grid=(N // BN,),
        in_specs=[
            pl.BlockSpec((BN, 1), lambda i: (i, 0)),
            pl.BlockSpec((BN, 1), lambda i: (i, 0)),
            pl.BlockSpec((BN, 1), lambda i: (i, 0)),
            _const_spec((128, DP)),
            _const_spec((1, DP)), _const_spec((1, DP)), _const_spec((1, DP)),
            _const_spec((DP, DP)), _const_spec((1, DP)),
        ],
        out_specs=[pl.BlockSpec((BN, H), lambda i: (i, 0)),
                   pl.BlockSpec((BN, H), lambda i: (i, 0))],
        out_shape=[jax.ShapeDtypeStruct((N, H), _f32),
                   jax.ShapeDtypeStruct((N, H), _f32)],
    )(z2, ch2, cg2, atp, w1a, w1b, b1, w2, b2)


def _edge_encode(a0, a1, a2, w1a, w1b, w1c, b1, w2, b2):
    return pl.pallas_call(
        _edge_body,
        grid=(E // BE,),
        in_specs=[
            pl.BlockSpec((BE, 1), lambda i: (i, 0)),
            pl.BlockSpec((BE, 1), lambda i: (i, 0)),
            pl.BlockSpec((BE, 1), lambda i: (i, 0)),
            _const_spec((1, DP)), _const_spec((1, DP)), _const_spec((1, DP)),
            _const_spec((1, DP)),
            _const_spec((DP, DP)), _const_spec((1, DP)),
        ],
        out_specs=[pl.BlockSpec((BE, H), lambda i: (i, 0)),
                   pl.BlockSpec((BE, H), lambda i: (i, 0))],
        out_shape=[jax.ShapeDtypeStruct((E, H), _f32),
                   jax.ShapeDtypeStruct((E, H), _f32)],
    )(a0, a1, a2, w1a, w1b, w1c, b1, w2, b2)


def _gin_mlp(x0, x1, agg0, agg1, w1, b1, w2, b2):
    return pl.pallas_call(
        _mlp_body,
        grid=(N // BN,),
        in_specs=[
            pl.BlockSpec((BN, H), lambda i: (i, 0)),
            pl.BlockSpec((BN, H), lambda i: (i, 0)),
            pl.BlockSpec((BN, H), lambda i: (i, 0)),
            pl.BlockSpec((BN, H), lambda i: (i, 0)),
            _const_spec((DP, DP)), _const_spec((1, DP)),
            _const_spec((DP, DP)), _const_spec((1, DP)),
        ],
        out_specs=[
            pl.BlockSpec((BN, DP), lambda i: (i, 0)),
            pl.BlockSpec((1, DP), lambda i: (0, 0)),
            pl.BlockSpec((1, DP), lambda i: (0, 0)),
        ],
        out_shape=[
            jax.ShapeDtypeStruct((N, DP), _f32),
            jax.ShapeDtypeStruct((1, DP), _f32),
            jax.ShapeDtypeStruct((1, DP), _f32),
        ],
    )(x0, x1, agg0, agg1, w1, b1, w2, b2)


def _bn_relu(h, s1, s2, g, b):
    return pl.pallas_call(
        _bn_body,
        grid=(N // BN,),
        in_specs=[
            pl.BlockSpec((BN, DP), lambda i: (i, 0)),
            _const_spec((1, DP)), _const_spec((1, DP)),
            _const_spec((1, DP)), _const_spec((1, DP)),
        ],
        out_specs=[
            pl.BlockSpec((BN, H), lambda i: (i, 0)),
            pl.BlockSpec((BN, H), lambda i: (i, 0)),
            pl.BlockSpec((1, DP), lambda i: (0, 0)),
        ],
        out_shape=[
            jax.ShapeDtypeStruct((N, H), _f32),
            jax.ShapeDtypeStruct((N, H), _f32),
            jax.ShapeDtypeStruct((1, DP), _f32),
        ],
    )(h, s1, s2, g, b)


def _pool(cs, w, b):
    return pl.pallas_call(
        _pool_body,
        in_specs=[_const_spec((1, DP)), _const_spec((DP, 300)),
                  _const_spec((1, 300))],
        out_specs=_const_spec((1, 300)),
        out_shape=jax.ShapeDtypeStruct((1, 300), _f32),
    )(cs, w, b)


def kernel(z, chirality, charge, edge_index, edge_attr, atom_table,
           np_W1, np_b1, np_W2, np_b2,
           ee_W1, ee_b1, ee_W2, ee_b2,
           mlp_W1, mlp_b1, mlp_W2, mlp_b2,
           bn_gamma, bn_beta, pool_W, pool_b):
    z2 = z.astype(jnp.int32).reshape(N, 1)
    ch2 = chirality.reshape(N, 1)
    cg2 = charge.reshape(N, 1)
    src = edge_index[0].astype(jnp.int32)
    dst = edge_index[1].astype(jnp.int32)
    a0 = edge_attr[:, 0:1]
    a1 = edge_attr[:, 1:2]
    a2 = edge_attr[:, 2:3]

    atp = _pad2(atom_table, (128, DP))
    np_w1a = _pad2(np_W1[0:1, :], (1, DP))
    np_w1b = _pad2(np_W1[1:2, :], (1, DP))
    np_b1p = _pad2(np_b1.reshape(1, -1), (1, DP))
    np_w2p = _pad2(np_W2, (DP, DP))
    np_b2p = _pad2(np_b2.reshape(1, -1), (1, DP))
    ee_w1a = _pad2(ee_W1[0:1, :], (1, DP))
    ee_w1b = _pad2(ee_W1[1:2, :], (1, DP))
    ee_w1c = _pad2(ee_W1[2:3, :], (1, DP))
    ee_b1p = _pad2(ee_b1.reshape(1, -1), (1, DP))
    ee_w2p = _pad2(ee_W2, (DP, DP))
    ee_b2p = _pad2(ee_b2.reshape(1, -1), (1, DP))
    w1p = _pad2(mlp_W1, (NLAYERS, DP, DP))
    b1p = _pad2(mlp_b1, (NLAYERS, DP))
    w2p = _pad2(mlp_W2, (NLAYERS, DP, DP))
    b2p = _pad2(mlp_b2, (NLAYERS, DP))
    gp = _pad2(bn_gamma, (NLAYERS, DP))
    bp = _pad2(bn_beta, (NLAYERS, DP))
    pwp = _pad2(pool_W, (DP, 300))
    pb2 = pool_b.reshape(1, 300)

    x0, x1 = _node_encode(z2, ch2, cg2, atp, np_w1a, np_w1b, np_b1p,
                          np_w2p, np_b2p)
    eah0, eah1 = _edge_encode(a0, a1, a2, ee_w1a, ee_w1b, ee_w1c, ee_b1p,
                              ee_w2p, ee_b2p)

    cs = None
    for i in range(NLAYERS):
        agg0, agg1 = _sc_aggregate(x0, x1, eah0, eah1, src, dst)
        h, s1, s2 = _gin_mlp(x0, x1, agg0, agg1, w1p[i],
                             b1p[i].reshape(1, DP),
                             w2p[i], b2p[i].reshape(1, DP))
        x0, x1, cs = _bn_relu(h, s1, s2, gp[i].reshape(1, DP),
                              bp[i].reshape(1, DP))
    return _pool(cs, pwp, pb2)


# speedup vs baseline: 1.1132x; 1.1132x over previous
import functools

import jax
import jax.numpy as jnp
from jax import lax
from jax.experimental import pallas as pl
from jax.experimental.pallas import tpu as pltpu
from jax.experimental.pallas import tpu_sc as plsc

N = 10000
E = 160000
DP = 320
H = DP // 2
NLAYERS = 5
NC = 2
NS = 16
EPT = E // NS
K = 40
NCHUNK = EPT // K
NRCH = N // K
BN = 400
BE = 800

_f32 = jnp.float32


def _pad2(a, shape):
    out = jnp.zeros(shape, a.dtype)
    return lax.dynamic_update_slice(out, a, (0,) * a.ndim)


def _const_spec(shape):
    nd = len(shape)
    return pl.BlockSpec(shape, lambda *args: (0,) * nd)


def _sc_body(xh0, xh1, eah0, eah1, src, dst, out0, out1, sh,
             xg0, xg1, eag0, eag1, isrc0, isrc1, idst0, idst1,
             semg0, semg1):
    c = lax.axis_index("c")
    s = lax.axis_index("s")

    nch = jnp.where(s < NRCH % NS, NRCH // NS + 1, NRCH // NS)

    def _zrow(r, carry):
        for i in range(H // 16):
            xg0[r, pl.ds(i * 16, 16)] = jnp.zeros((16,), _f32)
        return carry
    lax.fori_loop(0, K, _zrow, 0)

    def _zchunk(q, carry):
        base = (s + NS * q) * K
        pltpu.sync_copy(xg0, sh.at[pl.ds(base, K)])
        return carry
    lax.fori_loop(0, nch, _zchunk, 0)
    plsc.subcore_barrier()

    e0 = s * EPT
    bufs = ((xg0, eag0, isrc0, idst0, semg0),
            (xg1, eag1, isrc1, idst1, semg1))

    def _issue(j, buf):
        xg, eag, isrc, idst, semg = buf
        base = e0 + j * K
        pltpu.sync_copy(src.at[pl.ds(base, K)], isrc)
        pltpu.sync_copy(dst.at[pl.ds(base, K)], idst)
        esl = pl.ds(base, K)

        def _fire(xh, eah):
            def _go():
                pltpu.async_copy(xh.at[isrc], xg, semg)
                pltpu.async_copy(eah.at[esl], eag, semg)
            return _go
        pl.when(c == 0)(_fire(xh0, eah0))
        pl.when(c == 1)(_fire(xh1, eah1))

    def _finish(buf):
        xg, eag, isrc, idst, semg = buf
        pltpu.make_async_copy(xh0.at[pl.ds(0, K)], xg, semg).wait()
        pltpu.make_async_copy(eah0.at[pl.ds(0, K)], eag, semg).wait()

        def _mrow(r2, inner):
            r = r2 * 2
            for i in range(H // 16):
                sl = pl.ds(i * 16, 16)
                xg[r, sl] = jnp.maximum(xg[r, sl] + eag[r, sl], 0.0)
            for i in range(H // 16):
                sl = pl.ds(i * 16, 16)
                xg[r + 1, sl] = jnp.maximum(xg[r + 1, sl] + eag[r + 1, sl],
                                            0.0)
            return inner
        lax.fori_loop(0, K // 2, _mrow, 0)
        pltpu.sync_copy(xg, sh.at[idst], add=True)

    _issue(0, bufs[0])

    def _pair(j2, carry):
        j = j2 * 2
        _issue(j + 1, bufs[1])
        _finish(bufs[0])
        _issue(j + 2, bufs[0])
        _finish(bufs[1])
        return carry
    lax.fori_loop(0, NCHUNK // 2 - 1, _pair, 0)
    _issue(NCHUNK - 1, bufs[1])
    _finish(bufs[0])
    _finish(bufs[1])
    plsc.subcore_barrier()

    def _wchunk(q, carry):
        base = (s + NS * q) * K
        sl = pl.ds(base, K)
        pl.when(c == 0)(lambda: pltpu.sync_copy(sh.at[sl], out0.at[sl]))
        pl.when(c == 1)(lambda: pltpu.sync_copy(sh.at[sl], out1.at[sl]))
        return carry
    lax.fori_loop(0, nch, _wchunk, 0)


@functools.lru_cache(maxsize=1)
def _build_sc():
    mesh = plsc.VectorSubcoreMesh(
        core_axis_name="c", subcore_axis_name="s",
        num_cores=NC, num_subcores=NS)
    return pl.kernel(
        _sc_body,
        out_type=(jax.ShapeDtypeStruct((N, H), _f32),
                  jax.ShapeDtypeStruct((N, H), _f32)),
        mesh=mesh,
        scratch_types=[
            pltpu.VMEM_SHARED((N, H), _f32),
            pltpu.VMEM((K, H), _f32),
            pltpu.VMEM((K, H), _f32),
            pltpu.VMEM((K, H), _f32),
            pltpu.VMEM((K, H), _f32),
            pltpu.VMEM((K,), jnp.int32),
            pltpu.VMEM((K,), jnp.int32),
            pltpu.VMEM((K,), jnp.int32),
            pltpu.VMEM((K,), jnp.int32),
            pltpu.SemaphoreType.DMA,
            pltpu.SemaphoreType.DMA,
        ],
        compiler_params=pltpu.CompilerParams(use_tc_tiling_on_sc=False),
    )


def _sc_aggregate(xh0, xh1, eah0, eah1, src, dst):
    return _build_sc()(xh0, xh1, eah0, eah1, src, dst)


def _node_body(z_ref, ch_ref, cg_ref, at_ref, w1a_ref, w1b_ref, b1_ref,
               w2_ref, b2_ref, x0_ref, x1_ref):
    zb = z_ref[...]
    ids = lax.broadcasted_iota(jnp.int32, (BN, 128), 1)
    oh = (zb == ids).astype(_f32)
    emb = jnp.dot(oh, at_ref[...], preferred_element_type=_f32)
    t = ch_ref[...] * w1a_ref[...] + cg_ref[...] * w1b_ref[...] + b1_ref[...]
    t = jnp.maximum(t, 0.0)
    x = emb + jnp.dot(t, w2_ref[...],
                      preferred_element_type=_f32) + b2_ref[...]
    x0_ref[...] = x[:, 0:H]
    x1_ref[...] = x[:, H:DP]


def _edge_body(a0_ref, a1_ref, a2_ref, w1a_ref, w1b_ref, w1c_ref, b1_ref,
               w2_ref, b2_ref, o0_ref, o1_ref):
    t = (a0_ref[...] * w1a_ref[...] + a1_ref[...] * w1b_ref[...] +
         a2_ref[...] * w1c_ref[...] + b1_ref[...])
    t = jnp.maximum(t, 0.0)
    o = jnp.dot(t, w2_ref[...], preferred_element_type=_f32) + b2_ref[...]
    o0_ref[...] = o[:, 0:H]
    o1_ref[...] = o[:, H:DP]


def _mlp_body(x0_ref, x1_ref, a0_ref, a1_ref, w1_ref, b1_ref, w2_ref, b2_ref,
              h_ref, s1_ref, s2_ref):
    i = pl.program_id(0)
    t = jnp.dot(x0_ref[...] + a0_ref[...], w1_ref[0:H, :],
                preferred_element_type=_f32)
    t += jnp.dot(x1_ref[...] + a1_ref[...], w1_ref[H:DP, :],
                 preferred_element_type=_f32)
    t = jnp.maximum(t + b1_ref[...], 0.0)
    h = jnp.dot(t, w2_ref[...], preferred_element_type=_f32) + b2_ref[...]
    h_ref[...] = h

    @pl.when(i == 0)
    def _():
        s1_ref[...] = jnp.zeros_like(s1_ref)
        s2_ref[...] = jnp.zeros_like(s2_ref)
    s1_ref[...] += jnp.sum(h, axis=0, keepdims=True)
    s2_ref[...] += jnp.sum(h * h, axis=0, keepdims=True)


def _bn_body(h_ref, s1_ref, s2_ref, g_ref, b_ref, x0_ref, x1_ref, cs_ref):
    i = pl.program_id(0)
    mean = s1_ref[...] * (1.0 / N)
    var = s2_ref[...] * (1.0 / N) - mean * mean
    scale = g_ref[...] * lax.rsqrt(var + 1e-5)
    shift = b_ref[...] - mean * scale
    xb = jnp.maximum(h_ref[...] * scale + shift, 0.0)
    x0_ref[...] = xb[:, 0:H]
    x1_ref[...] = xb[:, H:DP]

    @pl.when(i == 0)
    def _():
        cs_ref[...] = jnp.zeros_like(cs_ref)
    cs_ref[...] += jnp.sum(xb, axis=0, keepdims=True)


def _pool_body(cs_ref, w_ref, b_ref, o_ref):
    o_ref[...] = jnp.dot(cs_ref[...] * (1.0 / N), w_ref[...],
                         preferred_element_type=_f32) + b_ref[...]


def _node_encode(z2, ch2, cg2, atp, w1a, w1b, b1, w2, b2):
    return pl.pallas_call(
        _node_body,
        grid=(N // BN,),
        in_specs=[
            pl.BlockSpec((BN, 1), lambda i: (i, 0)),
            pl.BlockSpec((BN, 1), lambda i: (i, 0)),
            pl.BlockSpec((BN, 1), lambda i: (i, 0)),
            _const_spec((128, DP)),
            _const_spec((1, DP)), _const_spec((1, DP)), _const_spec((1, DP)),
            _const_spec((DP, DP)), _const_spec((1, DP)),
        ],
        out_specs=[pl.BlockSpec((BN, H), lambda i: (i, 0)),
                   pl.BlockSpec((BN, H), lambda i: (i, 0))],
        out_shape=[jax.ShapeDtypeStruct((N, H), _f32),
                   jax.ShapeDtypeStruct((N, H), _f32)],
    )(z2, ch2, cg2, atp, w1a, w1b, b1, w2, b2)


def _edge_encode(a0, a1, a2, w1a, w1b, w1c, b1, w2, b2):
    return pl.pallas_call(
        _edge_body,
        grid=(E // BE,),
        in_specs=[
            pl.BlockSpec((BE, 1), lambda i: (i, 0)),
            pl.BlockSpec((BE, 1), lambda i: (i, 0)),
            pl.BlockSpec((BE, 1), lambda i: (i, 0)),
            _const_spec((1, DP)), _const_spec((1, DP)), _const_spec((1, DP)),
            _const_spec((1, DP)),
            _const_spec((DP, DP)), _const_spec((1, DP)),
        ],
        out_specs=[pl.BlockSpec((BE, H), lambda i: (i, 0)),
                   pl.BlockSpec((BE, H), lambda i: (i, 0))],
        out_shape=[jax.ShapeDtypeStruct((E, H), _f32),
                   jax.ShapeDtypeStruct((E, H), _f32)],
    )(a0, a1, a2, w1a, w1b, w1c, b1, w2, b2)


def _gin_mlp(x0, x1, agg0, agg1, w1, b1, w2, b2):
    return pl.pallas_call(
        _mlp_body,
        grid=(N // BN,),
        in_specs=[
            pl.BlockSpec((BN, H), lambda i: (i, 0)),
            pl.BlockSpec((BN, H), lambda i: (i, 0)),
            pl.BlockSpec((BN, H), lambda i: (i, 0)),
            pl.BlockSpec((BN, H), lambda i: (i, 0)),
            _const_spec((DP, DP)), _const_spec((1, DP)),
            _const_spec((DP, DP)), _const_spec((1, DP)),
        ],
        out_specs=[
            pl.BlockSpec((BN, DP), lambda i: (i, 0)),
            pl.BlockSpec((1, DP), lambda i: (0, 0)),
            pl.BlockSpec((1, DP), lambda i: (0, 0)),
        ],
        out_shape=[
            jax.ShapeDtypeStruct((N, DP), _f32),
            jax.ShapeDtypeStruct((1, DP), _f32),
            jax.ShapeDtypeStruct((1, DP), _f32),
        ],
    )(x0, x1, agg0, agg1, w1, b1, w2, b2)


def _bn_relu(h, s1, s2, g, b):
    return pl.pallas_call(
        _bn_body,
        grid=(N // BN,),
        in_specs=[
            pl.BlockSpec((BN, DP), lambda i: (i, 0)),
            _const_spec((1, DP)), _const_spec((1, DP)),
            _const_spec((1, DP)), _const_spec((1, DP)),
        ],
        out_specs=[
            pl.BlockSpec((BN, H), lambda i: (i, 0)),
            pl.BlockSpec((BN, H), lambda i: (i, 0)),
            pl.BlockSpec((1, DP), lambda i: (0, 0)),
        ],
        out_shape=[
            jax.ShapeDtypeStruct((N, H), _f32),
            jax.ShapeDtypeStruct((N, H), _f32),
            jax.ShapeDtypeStruct((1, DP), _f32),
        ],
    )(h, s1, s2, g, b)


def _pool(cs, w, b):
    return pl.pallas_call(
        _pool_body,
        in_specs=[_const_spec((1, DP)), _const_spec((DP, 300)),
                  _const_spec((1, 300))],
        out_specs=_const_spec((1, 300)),
        out_shape=jax.ShapeDtypeStruct((1, 300), _f32),
    )(cs, w, b)


def kernel(z, chirality, charge, edge_index, edge_attr, atom_table,
           np_W1, np_b1, np_W2, np_b2,
           ee_W1, ee_b1, ee_W2, ee_b2,
           mlp_W1, mlp_b1, mlp_W2, mlp_b2,
           bn_gamma, bn_beta, pool_W, pool_b):
    z2 = z.astype(jnp.int32).reshape(N, 1)
    ch2 = chirality.reshape(N, 1)
    cg2 = charge.reshape(N, 1)
    src = edge_index[0].astype(jnp.int32)
    dst = edge_index[1].astype(jnp.int32)
    a0 = edge_attr[:, 0:1]
    a1 = edge_attr[:, 1:2]
    a2 = edge_attr[:, 2:3]

    atp = _pad2(atom_table, (128, DP))
    np_w1a = _pad2(np_W1[0:1, :], (1, DP))
    np_w1b = _pad2(np_W1[1:2, :], (1, DP))
    np_b1p = _pad2(np_b1.reshape(1, -1), (1, DP))
    np_w2p = _pad2(np_W2, (DP, DP))
    np_b2p = _pad2(np_b2.reshape(1, -1), (1, DP))
    ee_w1a = _pad2(ee_W1[0:1, :], (1, DP))
    ee_w1b = _pad2(ee_W1[1:2, :], (1, DP))
    ee_w1c = _pad2(ee_W1[2:3, :], (1, DP))
    ee_b1p = _pad2(ee_b1.reshape(1, -1), (1, DP))
    ee_w2p = _pad2(ee_W2, (DP, DP))
    ee_b2p = _pad2(ee_b2.reshape(1, -1), (1, DP))
    w1p = _pad2(mlp_W1, (NLAYERS, DP, DP))
    b1p = _pad2(mlp_b1, (NLAYERS, DP))
    w2p = _pad2(mlp_W2, (NLAYERS, DP, DP))
    b2p = _pad2(mlp_b2, (NLAYERS, DP))
    gp = _pad2(bn_gamma, (NLAYERS, DP))
    bp = _pad2(bn_beta, (NLAYERS, DP))
    pwp = _pad2(pool_W, (DP, 300))
    pb2 = pool_b.reshape(1, 300)

    x0, x1 = _node_encode(z2, ch2, cg2, atp, np_w1a, np_w1b, np_b1p,
                          np_w2p, np_b2p)
    eah0, eah1 = _edge_encode(a0, a1, a2, ee_w1a, ee_w1b, ee_w1c, ee_b1p,
                              ee_w2p, ee_b2p)

    cs = None
    for i in range(NLAYERS):
        agg0, agg1 = _sc_aggregate(x0, x1, eah0, eah1, src, dst)
        h, s1, s2 = _gin_mlp(x0, x1, agg0, agg1, w1p[i],
                             b1p[i].reshape(1, DP),
                             w2p[i], b2p[i].reshape(1, DP))
        x0, x1, cs = _bn_relu(h, s1, s2, gp[i].reshape(1, DP),
                              bp[i].reshape(1, DP))
    return _pool(cs, pwp, pb2)


# speedup vs baseline: 1.3759x; 1.2360x over previous
import functools

import jax
import jax.numpy as jnp
from jax import lax
from jax.experimental import pallas as pl
from jax.experimental.pallas import tpu as pltpu
from jax.experimental.pallas import tpu_sc as plsc

N = 10000
E = 160000
DP = 320
H = DP // 2
NLAYERS = 5
NC = 2
NS = 16
EPT = E // NS
K = 40
NCHUNK = EPT // K
IG = 10
NG = NCHUNK // IG
NRCH = N // K
BN = 400
BE = 800

_f32 = jnp.float32


def _pad2(a, shape):
    out = jnp.zeros(shape, a.dtype)
    return lax.dynamic_update_slice(out, a, (0,) * a.ndim)


def _const_spec(shape):
    nd = len(shape)
    return pl.BlockSpec(shape, lambda *args: (0,) * nd)


def _sc_body(xh0, xh1, eah0, eah1, src, dst, out0, out1, sh,
             xg0, xg1, eag0, eag1, isg, idg, semg0, semg1):
    c = lax.axis_index("c")
    s = lax.axis_index("s")

    nch = jnp.where(s < NRCH % NS, NRCH // NS + 1, NRCH // NS)

    def _zrow(r, carry):
        for i in range(H // 16):
            xg0[r, pl.ds(i * 16, 16)] = jnp.zeros((16,), _f32)
        return carry
    lax.fori_loop(0, K, _zrow, 0)

    def _zchunk(q, carry):
        base = (s + NS * q) * K
        pltpu.sync_copy(xg0, sh.at[pl.ds(base, K)])
        return carry
    lax.fori_loop(0, nch, _zchunk, 0)
    plsc.subcore_barrier()

    e0 = s * EPT
    bufs = ((xg0, eag0, semg0), (xg1, eag1, semg1))

    def _ldgroup(g):
        gbase = e0 + g * IG * K
        gsel = g % 2
        pltpu.sync_copy(src.at[pl.ds(gbase, IG * K)], isg.at[gsel])
        pltpu.sync_copy(dst.at[pl.ds(gbase, IG * K)], idg.at[gsel])

    def _issue(j, buf):
        xg, eag, semg = buf
        gsel = (j // IG) % 2
        off = (j % IG) * K
        isv = isg.at[gsel, pl.ds(off, K)]
        esl = pl.ds(e0 + j * K, K)

        def _fire(xh, eah):
            def _go():
                pltpu.async_copy(xh.at[isv], xg, semg)
                pltpu.async_copy(eah.at[esl], eag, semg)
            return _go
        pl.when(c == 0)(_fire(xh0, eah0))
        pl.when(c == 1)(_fire(xh1, eah1))

    def _finish(j, buf):
        xg, eag, semg = buf
        pltpu.make_async_copy(xh0.at[pl.ds(0, K)], xg, semg).wait()
        pltpu.make_async_copy(eah0.at[pl.ds(0, K)], eag, semg).wait()

        def _mrow(r2, inner):
            r = r2 * 2
            for i in range(H // 16):
                sl = pl.ds(i * 16, 16)
                xg[r, sl] = jnp.maximum(xg[r, sl] + eag[r, sl], 0.0)
            for i in range(H // 16):
                sl = pl.ds(i * 16, 16)
                xg[r + 1, sl] = jnp.maximum(xg[r + 1, sl] + eag[r + 1, sl],
                                            0.0)
            return inner
        lax.fori_loop(0, K // 2, _mrow, 0)
        gsel = (j // IG) % 2
        off = (j % IG) * K
        pltpu.sync_copy(xg, sh.at[idg.at[gsel, pl.ds(off, K)]], add=True)

    _ldgroup(0)
    _issue(0, bufs[0])

    def _pair(j2, carry):
        j = j2 * 2
        g = j // IG
        pl.when(jnp.logical_and(j % IG == 0, g + 1 < NG))(
            lambda: _ldgroup(g + 1))
        _issue(j + 1, bufs[1])
        _finish(j, bufs[0])
        _issue(j + 2, bufs[0])
        _finish(j + 1, bufs[1])
        return carry
    lax.fori_loop(0, NCHUNK // 2 - 1, _pair, 0)
    _issue(NCHUNK - 1, bufs[1])
    _finish(NCHUNK - 2, bufs[0])
    _finish(NCHUNK - 1, bufs[1])
    plsc.subcore_barrier()

    def _wchunk(q, carry):
        base = (s + NS * q) * K
        sl = pl.ds(base, K)
        pl.when(c == 0)(lambda: pltpu.sync_copy(sh.at[sl], out0.at[sl]))
        pl.when(c == 1)(lambda: pltpu.sync_copy(sh.at[sl], out1.at[sl]))
        return carry
    lax.fori_loop(0, nch, _wchunk, 0)


@functools.lru_cache(maxsize=1)
def _build_sc():
    mesh = plsc.VectorSubcoreMesh(
        core_axis_name="c", subcore_axis_name="s",
        num_cores=NC, num_subcores=NS)
    return pl.kernel(
        _sc_body,
        out_type=(jax.ShapeDtypeStruct((N, H), _f32),
                  jax.ShapeDtypeStruct((N, H), _f32)),
        mesh=mesh,
        scratch_types=[
            pltpu.VMEM_SHARED((N, H), _f32),
            pltpu.VMEM((K, H), _f32),
            pltpu.VMEM((K, H), _f32),
            pltpu.VMEM((K, H), _f32),
            pltpu.VMEM((K, H), _f32),
            pltpu.VMEM((2, IG * K), jnp.int32),
            pltpu.VMEM((2, IG * K), jnp.int32),
            pltpu.SemaphoreType.DMA,
            pltpu.SemaphoreType.DMA,
        ],
        compiler_params=pltpu.CompilerParams(use_tc_tiling_on_sc=False),
    )


def _sc_aggregate(xh0, xh1, eah0, eah1, src, dst):
    return _build_sc()(xh0, xh1, eah0, eah1, src, dst)


def _node_body(z_ref, ch_ref, cg_ref, at_ref, w1a_ref, w1b_ref, b1_ref,
               w2_ref, b2_ref, x0_ref, x1_ref):
    zb = z_ref[...]
    ids = lax.broadcasted_iota(jnp.int32, (BN, 128), 1)
    oh = (zb == ids).astype(_f32)
    emb = jnp.dot(oh, at_ref[...], preferred_element_type=_f32)
    t = ch_ref[...] * w1a_ref[...] + cg_ref[...] * w1b_ref[...] + b1_ref[...]
    t = jnp.maximum(t, 0.0)
    x = emb + jnp.dot(t, w2_ref[...],
                      preferred_element_type=_f32) + b2_ref[...]
    x0_ref[...] = x[:, 0:H]
    x1_ref[...] = x[:, H:DP]


def _edge_body(a0_ref, a1_ref, a2_ref, w1a_ref, w1b_ref, w1c_ref, b1_ref,
               w2_ref, b2_ref, o0_ref, o1_ref):
    t = (a0_ref[...] * w1a_ref[...] + a1_ref[...] * w1b_ref[...] +
         a2_ref[...] * w1c_ref[...] + b1_ref[...])
    t = jnp.maximum(t, 0.0)
    o = jnp.dot(t, w2_ref[...], preferred_element_type=_f32) + b2_ref[...]
    o0_ref[...] = o[:, 0:H]
    o1_ref[...] = o[:, H:DP]


def _mlp_body(x0_ref, x1_ref, a0_ref, a1_ref, w1_ref, b1_ref, w2_ref, b2_ref,
              h_ref, s1_ref, s2_ref):
    i = pl.program_id(0)
    t = jnp.dot(x0_ref[...] + a0_ref[...], w1_ref[0:H, :],
                preferred_element_type=_f32)
    t += jnp.dot(x1_ref[...] + a1_ref[...], w1_ref[H:DP, :],
                 preferred_element_type=_f32)
    t = jnp.maximum(t + b1_ref[...], 0.0)
    h = jnp.dot(t, w2_ref[...], preferred_element_type=_f32) + b2_ref[...]
    h_ref[...] = h

    @pl.when(i == 0)
    def _():
        s1_ref[...] = jnp.zeros_like(s1_ref)
        s2_ref[...] = jnp.zeros_like(s2_ref)
    s1_ref[...] += jnp.sum(h, axis=0, keepdims=True)
    s2_ref[...] += jnp.sum(h * h, axis=0, keepdims=True)


def _bn_body(h_ref, s1_ref, s2_ref, g_ref, b_ref, x0_ref, x1_ref, cs_ref):
    i = pl.program_id(0)
    mean = s1_ref[...] * (1.0 / N)
    var = s2_ref[...] * (1.0 / N) - mean * mean
    scale = g_ref[...] * lax.rsqrt(var + 1e-5)
    shift = b_ref[...] - mean * scale
    xb = jnp.maximum(h_ref[...] * scale + shift, 0.0)
    x0_ref[...] = xb[:, 0:H]
    x1_ref[...] = xb[:, H:DP]

    @pl.when(i == 0)
    def _():
        cs_ref[...] = jnp.zeros_like(cs_ref)
    cs_ref[...] += jnp.sum(xb, axis=0, keepdims=True)


def _pool_body(cs_ref, w_ref, b_ref, o_ref):
    o_ref[...] = jnp.dot(cs_ref[...] * (1.0 / N), w_ref[...],
                         preferred_element_type=_f32) + b_ref[...]


def _node_encode(z2, ch2, cg2, atp, w1a, w1b, b1, w2, b2):
    return pl.pallas_call(
        _node_body,
        grid=(N // BN,),
        in_specs=[
            pl.BlockSpec((BN, 1), lambda i: (i, 0)),
            pl.BlockSpec((BN, 1), lambda i: (i, 0)),
            pl.BlockSpec((BN, 1), lambda i: (i, 0)),
            _const_spec((128, DP)),
            _const_spec((1, DP)), _const_spec((1, DP)), _const_spec((1, DP)),
            _const_spec((DP, DP)), _const_spec((1, DP)),
        ],
        out_specs=[pl.BlockSpec((BN, H), lambda i: (i, 0)),
                   pl.BlockSpec((BN, H), lambda i: (i, 0))],
        out_shape=[jax.ShapeDtypeStruct((N, H), _f32),
                   jax.ShapeDtypeStruct((N, H), _f32)],
    )(z2, ch2, cg2, atp, w1a, w1b, b1, w2, b2)


def _edge_encode(a0, a1, a2, w1a, w1b, w1c, b1, w2, b2):
    return pl.pallas_call(
        _edge_body,
        grid=(E // BE,),
        in_specs=[
            pl.BlockSpec((BE, 1), lambda i: (i, 0)),
            pl.BlockSpec((BE, 1), lambda i: (i, 0)),
            pl.BlockSpec((BE, 1), lambda i: (i, 0)),
            _const_spec((1, DP)), _const_spec((1, DP)), _const_spec((1, DP)),
            _const_spec((1, DP)),
            _const_spec((DP, DP)), _const_spec((1, DP)),
        ],
        out_specs=[pl.BlockSpec((BE, H), lambda i: (i, 0)),
                   pl.BlockSpec((BE, H), lambda i: (i, 0))],
        out_shape=[jax.ShapeDtypeStruct((E, H), _f32),
                   jax.ShapeDtypeStruct((E, H), _f32)],
    )(a0, a1, a2, w1a, w1b, w1c, b1, w2, b2)


def _gin_mlp(x0, x1, agg0, agg1, w1, b1, w2, b2):
    return pl.pallas_call(
        _mlp_body,
        grid=(N // BN,),
        in_specs=[
            pl.BlockSpec((BN, H), lambda i: (i, 0)),
            pl.BlockSpec((BN, H), lambda i: (i, 0)),
            pl.BlockSpec((BN, H), lambda i: (i, 0)),
            pl.BlockSpec((BN, H), lambda i: (i, 0)),
            _const_spec((DP, DP)), _const_spec((1, DP)),
            _const_spec((DP, DP)), _const_spec((1, DP)),
        ],
        out_specs=[
            pl.BlockSpec((BN, DP), lambda i: (i, 0)),
            pl.BlockSpec((1, DP), lambda i: (0, 0)),
            pl.BlockSpec((1, DP), lambda i: (0, 0)),
        ],
        out_shape=[
            jax.ShapeDtypeStruct((N, DP), _f32),
            jax.ShapeDtypeStruct((1, DP), _f32),
            jax.ShapeDtypeStruct((1, DP), _f32),
        ],
    )(x0, x1, agg0, agg1, w1, b1, w2, b2)


def _bn_relu(h, s1, s2, g, b):
    return pl.pallas_call(
        _bn_body,
        grid=(N // BN,),
        in_specs=[
            pl.BlockSpec((BN, DP), lambda i: (i, 0)),
            _const_spec((1, DP)), _const_spec((1, DP)),
            _const_spec((1, DP)), _const_spec((1, DP)),
        ],
        out_specs=[
            pl.BlockSpec((BN, H), lambda i: (i, 0)),
            pl.BlockSpec((BN, H), lambda i: (i, 0)),
            pl.BlockSpec((1, DP), lambda i: (0, 0)),
        ],
        out_shape=[
            jax.ShapeDtypeStruct((N, H), _f32),
            jax.ShapeDtypeStruct((N, H), _f32),
            jax.ShapeDtypeStruct((1, DP), _f32),
        ],
    )(h, s1, s2, g, b)


def _pool(cs, w, b):
    return pl.pallas_call(
        _pool_body,
        in_specs=[_const_spec((1, DP)), _const_spec((DP, 300)),
                  _const_spec((1, 300))],
        out_specs=_const_spec((1, 300)),
        out_shape=jax.ShapeDtypeStruct((1, 300), _f32),
    )(cs, w, b)


def kernel(z, chirality, charge, edge_index, edge_attr, atom_table,
           np_W1, np_b1, np_W2, np_b2,
           ee_W1, ee_b1, ee_W2, ee_b2,
           mlp_W1, mlp_b1, mlp_W2, mlp_b2,
           bn_gamma, bn_beta, pool_W, pool_b):
    z2 = z.astype(jnp.int32).reshape(N, 1)
    ch2 = chirality.reshape(N, 1)
    cg2 = charge.reshape(N, 1)
    src = edge_index[0].astype(jnp.int32)
    dst = edge_index[1].astype(jnp.int32)
    a0 = edge_attr[:, 0:1]
    a1 = edge_attr[:, 1:2]
    a2 = edge_attr[:, 2:3]

    atp = _pad2(atom_table, (128, DP))
    np_w1a = _pad2(np_W1[0:1, :], (1, DP))
    np_w1b = _pad2(np_W1[1:2, :], (1, DP))
    np_b1p = _pad2(np_b1.reshape(1, -1), (1, DP))
    np_w2p = _pad2(np_W2, (DP, DP))
    np_b2p = _pad2(np_b2.reshape(1, -1), (1, DP))
    ee_w1a = _pad2(ee_W1[0:1, :], (1, DP))
    ee_w1b = _pad2(ee_W1[1:2, :], (1, DP))
    ee_w1c = _pad2(ee_W1[2:3, :], (1, DP))
    ee_b1p = _pad2(ee_b1.reshape(1, -1), (1, DP))
    ee_w2p = _pad2(ee_W2, (DP, DP))
    ee_b2p = _pad2(ee_b2.reshape(1, -1), (1, DP))
    w1p = _pad2(mlp_W1, (NLAYERS, DP, DP))
    b1p = _pad2(mlp_b1, (NLAYERS, DP))
    w2p = _pad2(mlp_W2, (NLAYERS, DP, DP))
    b2p = _pad2(mlp_b2, (NLAYERS, DP))
    gp = _pad2(bn_gamma, (NLAYERS, DP))
    bp = _pad2(bn_beta, (NLAYERS, DP))
    pwp = _pad2(pool_W, (DP, 300))
    pb2 = pool_b.reshape(1, 300)

    x0, x1 = _node_encode(z2, ch2, cg2, atp, np_w1a, np_w1b, np_b1p,
                          np_w2p, np_b2p)
    eah0, eah1 = _edge_encode(a0, a1, a2, ee_w1a, ee_w1b, ee_w1c, ee_b1p,
                              ee_w2p, ee_b2p)

    cs = None
    for i in range(NLAYERS):
        agg0, agg1 = _sc_aggregate(x0, x1, eah0, eah1, src, dst)
        h, s1, s2 = _gin_mlp(x0, x1, agg0, agg1, w1p[i],
                             b1p[i].reshape(1, DP),
                             w2p[i], b2p[i].reshape(1, DP))
        x0, x1, cs = _bn_relu(h, s1, s2, gp[i].reshape(1, DP),
                              bp[i].reshape(1, DP))
    return _pool(cs, pwp, pb2)


# speedup vs baseline: 1.3860x; 1.0073x over previous
import functools

import jax
import jax.numpy as jnp
from jax import lax
from jax.experimental import pallas as pl
from jax.experimental.pallas import tpu as pltpu
from jax.experimental.pallas import tpu_sc as plsc

N = 10000
E = 160000
DP = 320
H = DP // 2
NLAYERS = 5
NC = 2
NS = 16
EPT = E // NS
K = 40
NCHUNK = EPT // K
IG = 10
NG = NCHUNK // IG
NRCH = N // K
BN = 400
BE = 800

_f32 = jnp.float32


def _pad2(a, shape):
    out = jnp.zeros(shape, a.dtype)
    return lax.dynamic_update_slice(out, a, (0,) * a.ndim)


def _const_spec(shape):
    nd = len(shape)
    return pl.BlockSpec(shape, lambda *args: (0,) * nd)


def _sc_body(xh0, xh1, eah0, eah1, src, dst, out0, out1, sh,
             xg0, xg1, eag0, eag1, isg, idg, semg0, semg1, semw):
    c = lax.axis_index("c")
    s = lax.axis_index("s")

    nch = jnp.where(s < NRCH % NS, NRCH // NS + 1, NRCH // NS)

    def _zrow(r, carry):
        for i in range(H // 16):
            xg0[r, pl.ds(i * 16, 16)] = jnp.zeros((16,), _f32)
        return carry
    lax.fori_loop(0, K, _zrow, 0)

    def _zchunk(q, carry):
        base = (s + NS * q) * K
        pltpu.async_copy(xg0, sh.at[pl.ds(base, K)], semw)
        return carry
    lax.fori_loop(0, nch, _zchunk, 0)

    def _zwait(q, carry):
        pltpu.make_async_copy(xg0, sh.at[pl.ds(0, K)], semw).wait()
        return carry
    lax.fori_loop(0, nch, _zwait, 0)
    plsc.subcore_barrier()

    e0 = s * EPT
    bufs = ((xg0, eag0, semg0), (xg1, eag1, semg1))

    def _ldgroup(g):
        gbase = e0 + g * IG * K
        gsel = g % 2
        pltpu.sync_copy(src.at[pl.ds(gbase, IG * K)], isg.at[gsel])
        pltpu.sync_copy(dst.at[pl.ds(gbase, IG * K)], idg.at[gsel])

    def _issue(j, buf):
        xg, eag, semg = buf
        gsel = (j // IG) % 2
        off = (j % IG) * K
        isv = isg.at[gsel, pl.ds(off, K)]
        esl = pl.ds(e0 + j * K, K)

        def _fire(xh, eah):
            def _go():
                pltpu.async_copy(xh.at[isv], xg, semg)
                pltpu.async_copy(eah.at[esl], eag, semg)
            return _go
        pl.when(c == 0)(_fire(xh0, eah0))
        pl.when(c == 1)(_fire(xh1, eah1))

    def _finish(j, buf):
        xg, eag, semg = buf
        pltpu.make_async_copy(xh0.at[pl.ds(0, K)], xg, semg).wait()
        pltpu.make_async_copy(eah0.at[pl.ds(0, K)], eag, semg).wait()

        def _mrow(r2, inner):
            r = r2 * 2
            for i in range(H // 16):
                sl = pl.ds(i * 16, 16)
                xg[r, sl] = jnp.maximum(xg[r, sl] + eag[r, sl], 0.0)
            for i in range(H // 16):
                sl = pl.ds(i * 16, 16)
                xg[r + 1, sl] = jnp.maximum(xg[r + 1, sl] + eag[r + 1, sl],
                                            0.0)
            return inner
        lax.fori_loop(0, K // 2, _mrow, 0)
        gsel = (j // IG) % 2
        off = (j % IG) * K
        pltpu.sync_copy(xg, sh.at[idg.at[gsel, pl.ds(off, K)]], add=True)

    _ldgroup(0)
    _issue(0, bufs[0])

    def _pair(j2, carry):
        j = j2 * 2
        g = j // IG
        pl.when(jnp.logical_and(j % IG == 0, g + 1 < NG))(
            lambda: _ldgroup(g + 1))
        _issue(j + 1, bufs[1])
        _finish(j, bufs[0])
        _issue(j + 2, bufs[0])
        _finish(j + 1, bufs[1])
        return carry
    lax.fori_loop(0, NCHUNK // 2 - 1, _pair, 0)
    _issue(NCHUNK - 1, bufs[1])
    _finish(NCHUNK - 2, bufs[0])
    _finish(NCHUNK - 1, bufs[1])
    plsc.subcore_barrier()

    def _wchunk(q, carry):
        base = (s + NS * q) * K
        sl = pl.ds(base, K)
        def _w0():
            pltpu.async_copy(sh.at[sl], out0.at[sl], semw)

        def _w1():
            pltpu.async_copy(sh.at[sl], out1.at[sl], semw)
        pl.when(c == 0)(_w0)
        pl.when(c == 1)(_w1)
        return carry
    lax.fori_loop(0, nch, _wchunk, 0)

    def _wwait(q, carry):
        pltpu.make_async_copy(sh.at[pl.ds(0, K)], out0.at[pl.ds(0, K)],
                              semw).wait()
        return carry
    lax.fori_loop(0, nch, _wwait, 0)


@functools.lru_cache(maxsize=1)
def _build_sc():
    mesh = plsc.VectorSubcoreMesh(
        core_axis_name="c", subcore_axis_name="s",
        num_cores=NC, num_subcores=NS)
    return pl.kernel(
        _sc_body,
        out_type=(jax.ShapeDtypeStruct((N, H), _f32),
                  jax.ShapeDtypeStruct((N, H), _f32)),
        mesh=mesh,
        scratch_types=[
            pltpu.VMEM_SHARED((N, H), _f32),
            pltpu.VMEM((K, H), _f32),
            pltpu.VMEM((K, H), _f32),
            pltpu.VMEM((K, H), _f32),
            pltpu.VMEM((K, H), _f32),
            pltpu.VMEM((2, IG * K), jnp.int32),
            pltpu.VMEM((2, IG * K), jnp.int32),
            pltpu.SemaphoreType.DMA,
            pltpu.SemaphoreType.DMA,
            pltpu.SemaphoreType.DMA,
        ],
        compiler_params=pltpu.CompilerParams(use_tc_tiling_on_sc=False),
    )


def _sc_aggregate(xh0, xh1, eah0, eah1, src, dst):
    return _build_sc()(xh0, xh1, eah0, eah1, src, dst)


def _node_body(z_ref, ch_ref, cg_ref, at_ref, w1a_ref, w1b_ref, b1_ref,
               w2_ref, b2_ref, x0_ref, x1_ref):
    zb = z_ref[...]
    ids = lax.broadcasted_iota(jnp.int32, (BN, 128), 1)
    oh = (zb == ids).astype(_f32)
    emb = jnp.dot(oh, at_ref[...], preferred_element_type=_f32)
    t = ch_ref[...] * w1a_ref[...] + cg_ref[...] * w1b_ref[...] + b1_ref[...]
    t = jnp.maximum(t, 0.0)
    x = emb + jnp.dot(t, w2_ref[...],
                      preferred_element_type=_f32) + b2_ref[...]
    x0_ref[...] = x[:, 0:H]
    x1_ref[...] = x[:, H:DP]


def _edge_body(a0_ref, a1_ref, a2_ref, w1a_ref, w1b_ref, w1c_ref, b1_ref,
               w2_ref, b2_ref, o0_ref, o1_ref):
    t = (a0_ref[...] * w1a_ref[...] + a1_ref[...] * w1b_ref[...] +
         a2_ref[...] * w1c_ref[...] + b1_ref[...])
    t = jnp.maximum(t, 0.0)
    o = jnp.dot(t, w2_ref[...], preferred_element_type=_f32) + b2_ref[...]
    o0_ref[...] = o[:, 0:H]
    o1_ref[...] = o[:, H:DP]


def _mlp_body(x0_ref, x1_ref, a0_ref, a1_ref, w1_ref, b1_ref, w2_ref, b2_ref,
              h_ref, s1_ref, s2_ref):
    i = pl.program_id(0)
    t = jnp.dot(x0_ref[...] + a0_ref[...], w1_ref[0:H, :],
                preferred_element_type=_f32)
    t += jnp.dot(x1_ref[...] + a1_ref[...], w1_ref[H:DP, :],
                 preferred_element_type=_f32)
    t = jnp.maximum(t + b1_ref[...], 0.0)
    h = jnp.dot(t, w2_ref[...], preferred_element_type=_f32) + b2_ref[...]
    h_ref[...] = h

    @pl.when(i == 0)
    def _():
        s1_ref[...] = jnp.zeros_like(s1_ref)
        s2_ref[...] = jnp.zeros_like(s2_ref)
    s1_ref[...] += jnp.sum(h, axis=0, keepdims=True)
    s2_ref[...] += jnp.sum(h * h, axis=0, keepdims=True)


def _bn_body(h_ref, s1_ref, s2_ref, g_ref, b_ref, x0_ref, x1_ref, cs_ref):
    i = pl.program_id(0)
    mean = s1_ref[...] * (1.0 / N)
    var = s2_ref[...] * (1.0 / N) - mean * mean
    scale = g_ref[...] * lax.rsqrt(var + 1e-5)
    shift = b_ref[...] - mean * scale
    xb = jnp.maximum(h_ref[...] * scale + shift, 0.0)
    x0_ref[...] = xb[:, 0:H]
    x1_ref[...] = xb[:, H:DP]

    @pl.when(i == 0)
    def _():
        cs_ref[...] = jnp.zeros_like(cs_ref)
    cs_ref[...] += jnp.sum(xb, axis=0, keepdims=True)


def _pool_body(cs_ref, w_ref, b_ref, o_ref):
    o_ref[...] = jnp.dot(cs_ref[...] * (1.0 / N), w_ref[...],
                         preferred_element_type=_f32) + b_ref[...]


def _node_encode(z2, ch2, cg2, atp, w1a, w1b, b1, w2, b2):
    return pl.pallas_call(
        _node_body,
        grid=(N // BN,),
        in_specs=[
            pl.BlockSpec((BN, 1), lambda i: (i, 0)),
            pl.BlockSpec((BN, 1), lambda i: (i, 0)),
            pl.BlockSpec((BN, 1), lambda i: (i, 0)),
            _const_spec((128, DP)),
            _const_spec((1, DP)), _const_spec((1, DP)), _const_spec((1, DP)),
            _const_spec((DP, DP)), _const_spec((1, DP)),
        ],
        out_specs=[pl.BlockSpec((BN, H), lambda i: (i, 0)),
                   pl.BlockSpec((BN, H), lambda i: (i, 0))],
        out_shape=[jax.ShapeDtypeStruct((N, H), _f32),
                   jax.ShapeDtypeStruct((N, H), _f32)],
    )(z2, ch2, cg2, atp, w1a, w1b, b1, w2, b2)


def _edge_encode(a0, a1, a2, w1a, w1b, w1c, b1, w2, b2):
    return pl.pallas_call(
        _edge_body,
        grid=(E // BE,),
        in_specs=[
            pl.BlockSpec((BE, 1), lambda i: (i, 0)),
            pl.BlockSpec((BE, 1), lambda i: (i, 0)),
            pl.BlockSpec((BE, 1), lambda i: (i, 0)),
            _const_spec((1, DP)), _const_spec((1, DP)), _const_spec((1, DP)),
            _const_spec((1, DP)),
            _const_spec((DP, DP)), _const_spec((1, DP)),
        ],
        out_specs=[pl.BlockSpec((BE, H), lambda i: (i, 0)),
                   pl.BlockSpec((BE, H), lambda i: (i, 0))],
        out_shape=[jax.ShapeDtypeStruct((E, H), _f32),
                   jax.ShapeDtypeStruct((E, H), _f32)],
    )(a0, a1, a2, w1a, w1b, w1c, b1, w2, b2)


def _gin_mlp(x0, x1, agg0, agg1, w1, b1, w2, b2):
    return pl.pallas_call(
        _mlp_body,
        grid=(N // BN,),
        in_specs=[
            pl.BlockSpec((BN, H), lambda i: (i, 0)),
            pl.BlockSpec((BN, H), lambda i: (i, 0)),
            pl.BlockSpec((BN, H), lambda i: (i, 0)),
            pl.BlockSpec((BN, H), lambda i: (i, 0)),
            _const_spec((DP, DP)), _const_spec((1, DP)),
            _const_spec((DP, DP)), _const_spec((1, DP)),
        ],
        out_specs=[
            pl.BlockSpec((BN, DP), lambda i: (i, 0)),
            pl.BlockSpec((1, DP), lambda i: (0, 0)),
            pl.BlockSpec((1, DP), lambda i: (0, 0)),
        ],
        out_shape=[
            jax.ShapeDtypeStruct((N, DP), _f32),
            jax.ShapeDtypeStruct((1, DP), _f32),
            jax.ShapeDtypeStruct((1, DP), _f32),
        ],
    )(x0, x1, agg0, agg1, w1, b1, w2, b2)


def _bn_relu(h, s1, s2, g, b):
    return pl.pallas_call(
        _bn_body,
        grid=(N // BN,),
        in_specs=[
            pl.BlockSpec((BN, DP), lambda i: (i, 0)),
            _const_spec((1, DP)), _const_spec((1, DP)),
            _const_spec((1, DP)), _const_spec((1, DP)),
        ],
        out_specs=[
            pl.BlockSpec((BN, H), lambda i: (i, 0)),
            pl.BlockSpec((BN, H), lambda i: (i, 0)),
            pl.BlockSpec((1, DP), lambda i: (0, 0)),
        ],
        out_shape=[
            jax.ShapeDtypeStruct((N, H), _f32),
            jax.ShapeDtypeStruct((N, H), _f32),
            jax.ShapeDtypeStruct((1, DP), _f32),
        ],
    )(h, s1, s2, g, b)


def _pool(cs, w, b):
    return pl.pallas_call(
        _pool_body,
        in_specs=[_const_spec((1, DP)), _const_spec((DP, 300)),
                  _const_spec((1, 300))],
        out_specs=_const_spec((1, 300)),
        out_shape=jax.ShapeDtypeStruct((1, 300), _f32),
    )(cs, w, b)


def kernel(z, chirality, charge, edge_index, edge_attr, atom_table,
           np_W1, np_b1, np_W2, np_b2,
           ee_W1, ee_b1, ee_W2, ee_b2,
           mlp_W1, mlp_b1, mlp_W2, mlp_b2,
           bn_gamma, bn_beta, pool_W, pool_b):
    z2 = z.astype(jnp.int32).reshape(N, 1)
    ch2 = chirality.reshape(N, 1)
    cg2 = charge.reshape(N, 1)
    src = edge_index[0].astype(jnp.int32)
    dst = edge_index[1].astype(jnp.int32)
    a0 = edge_attr[:, 0:1]
    a1 = edge_attr[:, 1:2]
    a2 = edge_attr[:, 2:3]

    atp = _pad2(atom_table, (128, DP))
    np_w1a = _pad2(np_W1[0:1, :], (1, DP))
    np_w1b = _pad2(np_W1[1:2, :], (1, DP))
    np_b1p = _pad2(np_b1.reshape(1, -1), (1, DP))
    np_w2p = _pad2(np_W2, (DP, DP))
    np_b2p = _pad2(np_b2.reshape(1, -1), (1, DP))
    ee_w1a = _pad2(ee_W1[0:1, :], (1, DP))
    ee_w1b = _pad2(ee_W1[1:2, :], (1, DP))
    ee_w1c = _pad2(ee_W1[2:3, :], (1, DP))
    ee_b1p = _pad2(ee_b1.reshape(1, -1), (1, DP))
    ee_w2p = _pad2(ee_W2, (DP, DP))
    ee_b2p = _pad2(ee_b2.reshape(1, -1), (1, DP))
    w1p = _pad2(mlp_W1, (NLAYERS, DP, DP))
    b1p = _pad2(mlp_b1, (NLAYERS, DP))
    w2p = _pad2(mlp_W2, (NLAYERS, DP, DP))
    b2p = _pad2(mlp_b2, (NLAYERS, DP))
    gp = _pad2(bn_gamma, (NLAYERS, DP))
    bp = _pad2(bn_beta, (NLAYERS, DP))
    pwp = _pad2(pool_W, (DP, 300))
    pb2 = pool_b.reshape(1, 300)

    x0, x1 = _node_encode(z2, ch2, cg2, atp, np_w1a, np_w1b, np_b1p,
                          np_w2p, np_b2p)
    eah0, eah1 = _edge_encode(a0, a1, a2, ee_w1a, ee_w1b, ee_w1c, ee_b1p,
                              ee_w2p, ee_b2p)

    cs = None
    for i in range(NLAYERS):
        agg0, agg1 = _sc_aggregate(x0, x1, eah0, eah1, src, dst)
        h, s1, s2 = _gin_mlp(x0, x1, agg0, agg1, w1p[i],
                             b1p[i].reshape(1, DP),
                             w2p[i], b2p[i].reshape(1, DP))
        x0, x1, cs = _bn_relu(h, s1, s2, gp[i].reshape(1, DP),
                              bp[i].reshape(1, DP))
    return _pool(cs, pwp, pb2)


# speedup vs baseline: 1.4839x; 1.0707x over previous
import functools

import jax
import jax.numpy as jnp
from jax import lax
from jax.experimental import pallas as pl
from jax.experimental.pallas import tpu as pltpu
from jax.experimental.pallas import tpu_sc as plsc

N = 10000
E = 160000
DP = 320
H = DP // 2
NLAYERS = 5
NC = 2
NS = 16
EPT = E // NS
K = 40
NCHUNK = EPT // K
IG = 10
NG = NCHUNK // IG
NRCH = N // K
BN = 2000
BE = 2000

_f32 = jnp.float32


def _pad2(a, shape):
    out = jnp.zeros(shape, a.dtype)
    return lax.dynamic_update_slice(out, a, (0,) * a.ndim)


def _const_spec(shape):
    nd = len(shape)
    return pl.BlockSpec(shape, lambda *args: (0,) * nd)


def _sc_body(xh0, xh1, eah0, eah1, src, dst, out0, out1, sh,
             xg0, xg1, eag0, eag1, isg, idg, semg0, semg1, semw):
    c = lax.axis_index("c")
    s = lax.axis_index("s")

    nch = jnp.where(s < NRCH % NS, NRCH // NS + 1, NRCH // NS)

    def _zrow(r, carry):
        for i in range(H // 16):
            xg0[r, pl.ds(i * 16, 16)] = jnp.zeros((16,), _f32)
        return carry
    lax.fori_loop(0, K, _zrow, 0)

    def _zchunk(q, carry):
        base = (s + NS * q) * K
        pltpu.async_copy(xg0, sh.at[pl.ds(base, K)], semw)
        return carry
    lax.fori_loop(0, nch, _zchunk, 0)

    def _zwait(q, carry):
        pltpu.make_async_copy(xg0, sh.at[pl.ds(0, K)], semw).wait()
        return carry
    lax.fori_loop(0, nch, _zwait, 0)
    plsc.subcore_barrier()

    e0 = s * EPT
    bufs = ((xg0, eag0, semg0), (xg1, eag1, semg1))

    def _ldgroup(g):
        gbase = e0 + g * IG * K
        gsel = g % 2
        pltpu.sync_copy(src.at[pl.ds(gbase, IG * K)], isg.at[gsel])
        pltpu.sync_copy(dst.at[pl.ds(gbase, IG * K)], idg.at[gsel])

    def _issue(j, buf):
        xg, eag, semg = buf
        gsel = (j // IG) % 2
        off = (j % IG) * K
        isv = isg.at[gsel, pl.ds(off, K)]
        esl = pl.ds(e0 + j * K, K)

        def _fire(xh, eah):
            def _go():
                pltpu.async_copy(xh.at[isv], xg, semg)
                pltpu.async_copy(eah.at[esl], eag, semg)
            return _go
        pl.when(c == 0)(_fire(xh0, eah0))
        pl.when(c == 1)(_fire(xh1, eah1))

    def _finish(j, buf):
        xg, eag, semg = buf
        pltpu.make_async_copy(xh0.at[pl.ds(0, K)], xg, semg).wait()
        pltpu.make_async_copy(eah0.at[pl.ds(0, K)], eag, semg).wait()

        def _mrow(r2, inner):
            r = r2 * 2
            for i in range(H // 16):
                sl = pl.ds(i * 16, 16)
                xg[r, sl] = jnp.maximum(xg[r, sl] + eag[r, sl], 0.0)
            for i in range(H // 16):
                sl = pl.ds(i * 16, 16)
                xg[r + 1, sl] = jnp.maximum(xg[r + 1, sl] + eag[r + 1, sl],
                                            0.0)
            return inner
        lax.fori_loop(0, K // 2, _mrow, 0)
        gsel = (j // IG) % 2
        off = (j % IG) * K
        pltpu.sync_copy(xg, sh.at[idg.at[gsel, pl.ds(off, K)]], add=True)

    _ldgroup(0)
    _issue(0, bufs[0])

    def _pair(j2, carry):
        j = j2 * 2
        g = j // IG
        pl.when(jnp.logical_and(j % IG == 0, g + 1 < NG))(
            lambda: _ldgroup(g + 1))
        _issue(j + 1, bufs[1])
        _finish(j, bufs[0])
        _issue(j + 2, bufs[0])
        _finish(j + 1, bufs[1])
        return carry
    lax.fori_loop(0, NCHUNK // 2 - 1, _pair, 0)
    _issue(NCHUNK - 1, bufs[1])
    _finish(NCHUNK - 2, bufs[0])
    _finish(NCHUNK - 1, bufs[1])
    plsc.subcore_barrier()

    def _wchunk(q, carry):
        base = (s + NS * q) * K
        sl = pl.ds(base, K)
        def _w0():
            pltpu.async_copy(sh.at[sl], out0.at[sl], semw)

        def _w1():
            pltpu.async_copy(sh.at[sl], out1.at[sl], semw)
        pl.when(c == 0)(_w0)
        pl.when(c == 1)(_w1)
        return carry
    lax.fori_loop(0, nch, _wchunk, 0)

    def _wwait(q, carry):
        pltpu.make_async_copy(sh.at[pl.ds(0, K)], out0.at[pl.ds(0, K)],
                              semw).wait()
        return carry
    lax.fori_loop(0, nch, _wwait, 0)


@functools.lru_cache(maxsize=1)
def _build_sc():
    mesh = plsc.VectorSubcoreMesh(
        core_axis_name="c", subcore_axis_name="s",
        num_cores=NC, num_subcores=NS)
    return pl.kernel(
        _sc_body,
        out_type=(jax.ShapeDtypeStruct((N, H), _f32),
                  jax.ShapeDtypeStruct((N, H), _f32)),
        mesh=mesh,
        scratch_types=[
            pltpu.VMEM_SHARED((N, H), _f32),
            pltpu.VMEM((K, H), _f32),
            pltpu.VMEM((K, H), _f32),
            pltpu.VMEM((K, H), _f32),
            pltpu.VMEM((K, H), _f32),
            pltpu.VMEM((2, IG * K), jnp.int32),
            pltpu.VMEM((2, IG * K), jnp.int32),
            pltpu.SemaphoreType.DMA,
            pltpu.SemaphoreType.DMA,
            pltpu.SemaphoreType.DMA,
        ],
        compiler_params=pltpu.CompilerParams(use_tc_tiling_on_sc=False),
    )


def _sc_aggregate(xh0, xh1, eah0, eah1, src, dst):
    return _build_sc()(xh0, xh1, eah0, eah1, src, dst)


def _node_body(z_ref, ch_ref, cg_ref, at_ref, w1a_ref, w1b_ref, b1_ref,
               w2_ref, b2_ref, x0_ref, x1_ref):
    zb = z_ref[...]
    ids = lax.broadcasted_iota(jnp.int32, (BN, 128), 1)
    oh = (zb == ids).astype(_f32)
    emb = jnp.dot(oh, at_ref[...], preferred_element_type=_f32)
    t = ch_ref[...] * w1a_ref[...] + cg_ref[...] * w1b_ref[...] + b1_ref[...]
    t = jnp.maximum(t, 0.0)
    x = emb + jnp.dot(t, w2_ref[...],
                      preferred_element_type=_f32) + b2_ref[...]
    x0_ref[...] = x[:, 0:H]
    x1_ref[...] = x[:, H:DP]


def _edge_body(a0_ref, a1_ref, a2_ref, w1a_ref, w1b_ref, w1c_ref, b1_ref,
               w2_ref, b2_ref, o0_ref, o1_ref):
    t = (a0_ref[...] * w1a_ref[...] + a1_ref[...] * w1b_ref[...] +
         a2_ref[...] * w1c_ref[...] + b1_ref[...])
    t = jnp.maximum(t, 0.0)
    o = jnp.dot(t, w2_ref[...], preferred_element_type=_f32) + b2_ref[...]
    o0_ref[...] = o[:, 0:H]
    o1_ref[...] = o[:, H:DP]


def _mlp_body(x0_ref, x1_ref, a0_ref, a1_ref, w1_ref, b1_ref, w2_ref, b2_ref,
              h_ref, s1_ref, s2_ref):
    i = pl.program_id(0)
    t = jnp.dot(x0_ref[...] + a0_ref[...], w1_ref[0:H, :],
                preferred_element_type=_f32)
    t += jnp.dot(x1_ref[...] + a1_ref[...], w1_ref[H:DP, :],
                 preferred_element_type=_f32)
    t = jnp.maximum(t + b1_ref[...], 0.0)
    h = jnp.dot(t, w2_ref[...], preferred_element_type=_f32) + b2_ref[...]
    h_ref[...] = h

    @pl.when(i == 0)
    def _():
        s1_ref[...] = jnp.zeros_like(s1_ref)
        s2_ref[...] = jnp.zeros_like(s2_ref)
    s1_ref[...] += jnp.sum(h, axis=0, keepdims=True)
    s2_ref[...] += jnp.sum(h * h, axis=0, keepdims=True)


def _bn_body(h_ref, s1_ref, s2_ref, g_ref, b_ref, x0_ref, x1_ref, cs_ref):
    i = pl.program_id(0)
    mean = s1_ref[...] * (1.0 / N)
    var = s2_ref[...] * (1.0 / N) - mean * mean
    scale = g_ref[...] * lax.rsqrt(var + 1e-5)
    shift = b_ref[...] - mean * scale
    xb = jnp.maximum(h_ref[...] * scale + shift, 0.0)
    x0_ref[...] = xb[:, 0:H]
    x1_ref[...] = xb[:, H:DP]

    @pl.when(i == 0)
    def _():
        cs_ref[...] = jnp.zeros_like(cs_ref)
    cs_ref[...] += jnp.sum(xb, axis=0, keepdims=True)


def _pool_body(cs_ref, w_ref, b_ref, o_ref):
    o_ref[...] = jnp.dot(cs_ref[...] * (1.0 / N), w_ref[...],
                         preferred_element_type=_f32) + b_ref[...]


def _node_encode(z2, ch2, cg2, atp, w1a, w1b, b1, w2, b2):
    return pl.pallas_call(
        _node_body,
        grid=(N // BN,),
        in_specs=[
            pl.BlockSpec((BN, 1), lambda i: (i, 0)),
            pl.BlockSpec((BN, 1), lambda i: (i, 0)),
            pl.BlockSpec((BN, 1), lambda i: (i, 0)),
            _const_spec((128, DP)),
            _const_spec((1, DP)), _const_spec((1, DP)), _const_spec((1, DP)),
            _const_spec((DP, DP)), _const_spec((1, DP)),
        ],
        out_specs=[pl.BlockSpec((BN, H), lambda i: (i, 0)),
                   pl.BlockSpec((BN, H), lambda i: (i, 0))],
        out_shape=[jax.ShapeDtypeStruct((N, H), _f32),
                   jax.ShapeDtypeStruct((N, H), _f32)],
    )(z2, ch2, cg2, atp, w1a, w1b, b1, w2, b2)


def _edge_encode(a0, a1, a2, w1a, w1b, w1c, b1, w2, b2):
    return pl.pallas_call(
        _edge_body,
        grid=(E // BE,),
        in_specs=[
            pl.BlockSpec((BE, 1), lambda i: (i, 0)),
            pl.BlockSpec((BE, 1), lambda i: (i, 0)),
            pl.BlockSpec((BE, 1), lambda i: (i, 0)),
            _const_spec((1, DP)), _const_spec((1, DP)), _const_spec((1, DP)),
            _const_spec((1, DP)),
            _const_spec((DP, DP)), _const_spec((1, DP)),
        ],
        out_specs=[pl.BlockSpec((BE, H), lambda i: (i, 0)),
                   pl.BlockSpec((BE, H), lambda i: (i, 0))],
        out_shape=[jax.ShapeDtypeStruct((E, H), _f32),
                   jax.ShapeDtypeStruct((E, H), _f32)],
    )(a0, a1, a2, w1a, w1b, w1c, b1, w2, b2)


def _gin_mlp(x0, x1, agg0, agg1, w1, b1, w2, b2):
    return pl.pallas_call(
        _mlp_body,
        grid=(N // BN,),
        in_specs=[
            pl.BlockSpec((BN, H), lambda i: (i, 0)),
            pl.BlockSpec((BN, H), lambda i: (i, 0)),
            pl.BlockSpec((BN, H), lambda i: (i, 0)),
            pl.BlockSpec((BN, H), lambda i: (i, 0)),
            _const_spec((DP, DP)), _const_spec((1, DP)),
            _const_spec((DP, DP)), _const_spec((1, DP)),
        ],
        out_specs=[
            pl.BlockSpec((BN, DP), lambda i: (i, 0)),
            pl.BlockSpec((1, DP), lambda i: (0, 0)),
            pl.BlockSpec((1, DP), lambda i: (0, 0)),
        ],
        out_shape=[
            jax.ShapeDtypeStruct((N, DP), _f32),
            jax.ShapeDtypeStruct((1, DP), _f32),
            jax.ShapeDtypeStruct((1, DP), _f32),
        ],
    )(x0, x1, agg0, agg1, w1, b1, w2, b2)


def _bn_relu(h, s1, s2, g, b):
    return pl.pallas_call(
        _bn_body,
        grid=(N // BN,),
        in_specs=[
            pl.BlockSpec((BN, DP), lambda i: (i, 0)),
            _const_spec((1, DP)), _const_spec((1, DP)),
            _const_spec((1, DP)), _const_spec((1, DP)),
        ],
        out_specs=[
            pl.BlockSpec((BN, H), lambda i: (i, 0)),
            pl.BlockSpec((BN, H), lambda i: (i, 0)),
            pl.BlockSpec((1, DP), lambda i: (0, 0)),
        ],
        out_shape=[
            jax.ShapeDtypeStruct((N, H), _f32),
            jax.ShapeDtypeStruct((N, H), _f32),
            jax.ShapeDtypeStruct((1, DP), _f32),
        ],
    )(h, s1, s2, g, b)


def _pool(cs, w, b):
    return pl.pallas_call(
        _pool_body,
        in_specs=[_const_spec((1, DP)), _const_spec((DP, 300)),
                  _const_spec((1, 300))],
        out_specs=_const_spec((1, 300)),
        out_shape=jax.ShapeDtypeStruct((1, 300), _f32),
    )(cs, w, b)


def kernel(z, chirality, charge, edge_index, edge_attr, atom_table,
           np_W1, np_b1, np_W2, np_b2,
           ee_W1, ee_b1, ee_W2, ee_b2,
           mlp_W1, mlp_b1, mlp_W2, mlp_b2,
           bn_gamma, bn_beta, pool_W, pool_b):
    z2 = z.astype(jnp.int32).reshape(N, 1)
    ch2 = chirality.reshape(N, 1)
    cg2 = charge.reshape(N, 1)
    src = edge_index[0].astype(jnp.int32)
    dst = edge_index[1].astype(jnp.int32)
    a0 = edge_attr[:, 0:1]
    a1 = edge_attr[:, 1:2]
    a2 = edge_attr[:, 2:3]

    atp = _pad2(atom_table, (128, DP))
    np_w1a = _pad2(np_W1[0:1, :], (1, DP))
    np_w1b = _pad2(np_W1[1:2, :], (1, DP))
    np_b1p = _pad2(np_b1.reshape(1, -1), (1, DP))
    np_w2p = _pad2(np_W2, (DP, DP))
    np_b2p = _pad2(np_b2.reshape(1, -1), (1, DP))
    ee_w1a = _pad2(ee_W1[0:1, :], (1, DP))
    ee_w1b = _pad2(ee_W1[1:2, :], (1, DP))
    ee_w1c = _pad2(ee_W1[2:3, :], (1, DP))
    ee_b1p = _pad2(ee_b1.reshape(1, -1), (1, DP))
    ee_w2p = _pad2(ee_W2, (DP, DP))
    ee_b2p = _pad2(ee_b2.reshape(1, -1), (1, DP))
    w1p = _pad2(mlp_W1, (NLAYERS, DP, DP))
    b1p = _pad2(mlp_b1, (NLAYERS, DP))
    w2p = _pad2(mlp_W2, (NLAYERS, DP, DP))
    b2p = _pad2(mlp_b2, (NLAYERS, DP))
    gp = _pad2(bn_gamma, (NLAYERS, DP))
    bp = _pad2(bn_beta, (NLAYERS, DP))
    pwp = _pad2(pool_W, (DP, 300))
    pb2 = pool_b.reshape(1, 300)

    x0, x1 = _node_encode(z2, ch2, cg2, atp, np_w1a, np_w1b, np_b1p,
                          np_w2p, np_b2p)
    eah0, eah1 = _edge_encode(a0, a1, a2, ee_w1a, ee_w1b, ee_w1c, ee_b1p,
                              ee_w2p, ee_b2p)

    cs = None
    for i in range(NLAYERS):
        agg0, agg1 = _sc_aggregate(x0, x1, eah0, eah1, src, dst)
        h, s1, s2 = _gin_mlp(x0, x1, agg0, agg1, w1p[i],
                             b1p[i].reshape(1, DP),
                             w2p[i], b2p[i].reshape(1, DP))
        x0, x1, cs = _bn_relu(h, s1, s2, gp[i].reshape(1, DP),
                              bp[i].reshape(1, DP))
    return _pool(cs, pwp, pb2)
